# Initial kernel scaffold; baseline (speedup 1.0000x reference)
import jax
import jax.numpy as jnp
from jax.experimental import pallas as pl


def kernel(indices, weight):
    raise NotImplementedError("write your pallas kernel here")


# speedup vs baseline: 1.6420x; 1.6420x over previous
import functools

import jax
import jax.numpy as jnp
from jax import lax
from jax.experimental import pallas as pl
from jax.experimental.pallas import tpu as pltpu
from jax.experimental.pallas import tpu_sc as plsc

BATCH, HIST, DIM = 4096, 200, 128
TOTAL = BATCH * HIST


@functools.partial(jax.jit, static_argnames=())
def _embed(indices_flat, weight):
    info = plsc.get_sparse_core_info()
    nw = info.num_cores * info.num_subcores
    per_w = TOTAL // nw
    chunk = 512
    n_chunks = per_w // chunk

    mesh = plsc.VectorSubcoreMesh(core_axis_name="c", subcore_axis_name="s")

    @functools.partial(
        pl.kernel,
        mesh=mesh,
        out_type=jax.ShapeDtypeStruct((TOTAL, DIM), jnp.float32),
        scratch_types=[
            pltpu.VMEM((chunk,), jnp.int32),
            pltpu.VMEM((chunk, DIM), jnp.float32),
            pltpu.SemaphoreType.DMA,
        ],
    )
    def k(idx_hbm, table_hbm, out_hbm, idx_v, rows_v, sem):
        wid = lax.axis_index("s") * info.num_cores + lax.axis_index("c")
        base = wid * per_w

        def body(i, carry):
            off = base + i * chunk
            pltpu.sync_copy(idx_hbm.at[pl.ds(off, chunk)], idx_v)
            pltpu.async_copy(table_hbm.at[idx_v], rows_v, sem).wait()
            pltpu.sync_copy(rows_v, out_hbm.at[pl.ds(off, chunk)])
            return carry

        lax.fori_loop(0, n_chunks, body, 0)

    return k(indices_flat, weight)


def kernel(indices, weight):
    flat = indices.reshape(-1).astype(jnp.int32)
    out = _embed(flat, weight)
    return out.reshape(BATCH, HIST, DIM)


# speedup vs baseline: 1.8458x; 1.1241x over previous
import functools

import jax
import jax.numpy as jnp
from jax import lax
from jax.experimental import pallas as pl
from jax.experimental.pallas import tpu as pltpu
from jax.experimental.pallas import tpu_sc as plsc

BATCH, HIST, DIM = 4096, 200, 128
TOTAL = BATCH * HIST


@functools.partial(jax.jit, static_argnames=())
def _embed(indices_flat, weight):
    info = plsc.get_sparse_core_info()
    nw = info.num_cores * info.num_subcores
    per_w = TOTAL // nw
    chunk = 400
    n_chunks = per_w // chunk
    n_groups = n_chunks // 2

    mesh = plsc.VectorSubcoreMesh(core_axis_name="c", subcore_axis_name="s")

    @functools.partial(
        pl.kernel,
        mesh=mesh,
        out_type=jax.ShapeDtypeStruct((TOTAL, DIM), jnp.float32),
        scratch_types=[
            pltpu.VMEM((chunk,), jnp.int32),
            pltpu.VMEM((chunk,), jnp.int32),
            pltpu.VMEM((chunk, DIM), jnp.float32),
            pltpu.VMEM((chunk, DIM), jnp.float32),
            pltpu.SemaphoreType.DMA,
            pltpu.SemaphoreType.DMA,
            pltpu.SemaphoreType.DMA,
            pltpu.SemaphoreType.DMA,
        ],
    )
    def k(idx_hbm, table_hbm, out_hbm, idx0, idx1, rows0, rows1, g0, g1, w0, w1):
        idxs, rows, gs, ws = (idx0, idx1), (rows0, rows1), (g0, g1), (w0, w1)
        wid = lax.axis_index("s") * info.num_cores + lax.axis_index("c")
        base = wid * per_w
        last = base + per_w - chunk

        def idx_load(b, off):
            pltpu.sync_copy(idx_hbm.at[pl.ds(off, chunk)], idxs[b])

        def gather_start(b):
            pltpu.async_copy(table_hbm.at[idxs[b]], rows[b], gs[b])

        def gather_wait(b):
            pltpu.make_async_copy(table_hbm.at[idxs[b]], rows[b], gs[b]).wait()

        def wb_start(b, off):
            pltpu.async_copy(rows[b], out_hbm.at[pl.ds(off, chunk)], ws[b])

        def wb_wait(b):
            pltpu.make_async_copy(rows[b], out_hbm.at[pl.ds(0, chunk)], ws[b]).wait()

        for b in (0, 1):
            idx_load(b, base + b * chunk)
            gather_start(b)

        def group(g, carry):
            for b in (0, 1):
                off = base + (2 * g + b) * chunk
                gather_wait(b)
                wb_start(b, off)
                nxt = jnp.minimum(off + 2 * chunk, last)
                idx_load(b, nxt)
                wb_wait(b)
                gather_start(b)
            return carry

        lax.fori_loop(0, n_groups, group, 0)

        for b in (0, 1):
            gather_wait(b)

    return k(indices_flat, weight)


def kernel(indices, weight):
    flat = indices.reshape(-1).astype(jnp.int32)
    out = _embed(flat, weight)
    return out.reshape(BATCH, HIST, DIM)


# speedup vs baseline: 1.8479x; 1.0012x over previous
import functools

import jax
import jax.numpy as jnp
from jax import lax
from jax.experimental import pallas as pl
from jax.experimental.pallas import tpu as pltpu
from jax.experimental.pallas import tpu_sc as plsc

BATCH, HIST, DIM = 4096, 200, 128
TOTAL = BATCH * HIST


@functools.partial(jax.jit, static_argnames=())
def _embed(indices_flat, weight):
    info = plsc.get_sparse_core_info()
    nw = info.num_cores * info.num_subcores
    per_w = TOTAL // nw
    chunk = 320
    n_chunks = per_w // chunk
    n_groups = n_chunks // 2

    mesh = plsc.VectorSubcoreMesh(core_axis_name="c", subcore_axis_name="s")

    @functools.partial(
        pl.kernel,
        mesh=mesh,
        out_type=jax.ShapeDtypeStruct((TOTAL, DIM), jnp.float32),
        scratch_types=[
            pltpu.VMEM((per_w,), jnp.int32),
            pltpu.VMEM((chunk, DIM), jnp.float32),
            pltpu.VMEM((chunk, DIM), jnp.float32),
            pltpu.SemaphoreType.DMA,
            pltpu.SemaphoreType.DMA,
            pltpu.SemaphoreType.DMA,
            pltpu.SemaphoreType.DMA,
        ],
    )
    def k(idx_hbm, table_hbm, out_hbm, idx_v, rows0, rows1, g0, g1, w0, w1):
        rows, gs, ws = (rows0, rows1), (g0, g1), (w0, w1)
        wid = lax.axis_index("s") * info.num_cores + lax.axis_index("c")
        base = wid * per_w
        last_i = n_chunks - 1

        pltpu.sync_copy(idx_hbm.at[pl.ds(base, per_w)], idx_v)

        def idx_slice(i):
            return idx_v.at[pl.ds(pl.multiple_of(i * chunk, 8), chunk)]

        def gather_start(b, i):
            pltpu.async_copy(table_hbm.at[idx_slice(i)], rows[b], gs[b])

        def gather_wait(b, i):
            pltpu.make_async_copy(table_hbm.at[idx_slice(i)], rows[b], gs[b]).wait()

        def wb_start(b, i):
            pltpu.async_copy(rows[b], out_hbm.at[pl.ds(base + i * chunk, chunk)], ws[b])

        def wb_wait(b):
            pltpu.make_async_copy(rows[b], out_hbm.at[pl.ds(0, chunk)], ws[b]).wait()

        for b in (0, 1):
            gather_start(b, b)

        def group(g, carry):
            for b in (0, 1):
                i = 2 * g + b
                gather_wait(b, i)
                wb_start(b, i)
                nxt = jnp.minimum(i + 2, last_i)
                wb_wait(b)
                gather_start(b, nxt)
            return carry

        lax.fori_loop(0, n_groups, group, 0)

        for b in (0, 1):
            gather_wait(b, last_i)

    return k(indices_flat, weight)


def kernel(indices, weight):
    flat = indices.reshape(-1).astype(jnp.int32)
    out = _embed(flat, weight)
    return out.reshape(BATCH, HIST, DIM)
